# baseline (speedup 1.0000x reference)
import functools

import jax
import jax.numpy as jnp
from jax.experimental import pallas as pl

_BATCH_BLOCK = 256
_HIST = 50
_WIDTH = 74


def _onehot_body(h_ref, dw_ref, dm_ref, mo_ref, o_ref):
    shape = o_ref.shape
    lane = jax.lax.broadcasted_iota(jnp.int32, shape, 2)
    h = jnp.clip(h_ref[...], 0, 23)[:, :, None]
    dw = jnp.clip(dw_ref[...], 0, 6)[:, :, None] + 24
    dm = jnp.clip(dm_ref[...], 0, 30)[:, :, None] + 31
    mo = jnp.clip(mo_ref[...], 0, 11)[:, :, None] + 62
    hit = (lane == h) | (lane == dw) | (lane == dm) | (lane == mo)
    o_ref[...] = hit.astype(jnp.float32)


@functools.partial(jax.jit, static_argnums=())
def kernel(hour, day_of_week, day_of_month, month):
    b, hist = hour.shape
    grid = (b // _BATCH_BLOCK,)
    in_spec = pl.BlockSpec((_BATCH_BLOCK, hist), lambda i: (i, 0))
    out_spec = pl.BlockSpec((_BATCH_BLOCK, hist, _WIDTH), lambda i: (i, 0, 0))
    args = [x.astype(jnp.int32) for x in (hour, day_of_week, day_of_month, month)]
    return pl.pallas_call(
        _onehot_body,
        grid=grid,
        in_specs=[in_spec] * 4,
        out_specs=out_spec,
        out_shape=jax.ShapeDtypeStruct((b, hist, _WIDTH), jnp.float32),
    )(*args)
